# baseline (speedup 1.0000x reference)
import functools

import jax
import jax.numpy as jnp
from jax import lax
from jax.experimental import pallas as pl
from jax.experimental.pallas import tpu as pltpu
from jax.experimental.pallas import tpu_sc as plsc

N_NODES = 10000
D = 128
N_EDGES = 320000

NC = 2
NS = 16
N_TILES = NC * NS
CHUNK = 64
N_CHUNKS = 160
EPT_PAD = N_CHUNKS * CHUNK
E_PAD = EPT_PAD * N_TILES
ROWS_PER_TILE = 640
N_STAGE = 10
N_PAD = ROWS_PER_TILE * NS
TRASH_ROW = N_NODES
DEG_W = 16


def _agg_body(p_hbm, src_hbm, dst_hbm, zfeat,
              acc_out, acc_sh, src_b, dst_b, rows_b, sem, sem2):
    c = lax.axis_index("c")
    s = lax.axis_index("s")
    g = c * NS + s
    r0 = s * ROWS_PER_TILE

    pltpu.sync_copy(zfeat.at[pl.ds(pl.multiple_of(r0, CHUNK), CHUNK)],
                    rows_b.at[0])

    def zinit(k, carry):
        rk = pl.multiple_of(r0 + k * CHUNK, CHUNK)
        pltpu.async_copy(rows_b.at[0], acc_sh.at[pl.ds(rk, CHUNK)],
                         sem2.at[0])
        return carry

    lax.fori_loop(0, N_STAGE, zinit, 0)

    def zdrain(k, carry):
        pltpu.make_async_copy(rows_b.at[0],
                              acc_sh.at[pl.ds(pl.multiple_of(r0, CHUNK),
                                              CHUNK)], sem2.at[0]).wait()
        return carry

    lax.fori_loop(0, N_STAGE, zdrain, 0)
    plsc.subcore_barrier()
    base = g * EPT_PAD

    pltpu.sync_copy(src_hbm.at[pl.ds(pl.multiple_of(base, CHUNK), CHUNK)],
                    src_b.at[0])
    pltpu.async_copy(p_hbm.at[src_b.at[0]], rows_b.at[0], sem.at[0])

    def body(i, carry):
        j = lax.rem(i, 4)
        jn = lax.rem(i + 1, 4)

        @pl.when(i >= 3)
        def _drain_scatter():
            pltpu.make_async_copy(rows_b.at[jn], acc_sh.at[dst_b.at[jn]],
                                  sem2.at[jn]).wait()

        @pl.when(i < N_CHUNKS - 1)
        def _prefetch():
            off = pl.multiple_of(base + (i + 1) * CHUNK, CHUNK)
            pltpu.sync_copy(src_hbm.at[pl.ds(off, CHUNK)], src_b.at[jn])
            pltpu.async_copy(p_hbm.at[src_b.at[jn]], rows_b.at[jn], sem.at[jn])

        off_d = pl.multiple_of(base + i * CHUNK, CHUNK)
        pltpu.sync_copy(dst_hbm.at[pl.ds(off_d, CHUNK)], dst_b.at[j])
        pltpu.make_async_copy(p_hbm.at[src_b.at[j]], rows_b.at[j], sem.at[j]).wait()
        pltpu.async_copy(rows_b.at[j], acc_sh.at[dst_b.at[j]], sem2.at[j], add=True)
        return carry

    lax.fori_loop(0, N_CHUNKS, body, 0)
    def fdrain(t, carry):
        jt = lax.rem(N_CHUNKS - 1 - t, 4)
        pltpu.make_async_copy(rows_b.at[jt], acc_sh.at[dst_b.at[jt]],
                              sem2.at[jt]).wait()
        return carry

    lax.fori_loop(0, 3, fdrain, 0)
    plsc.subcore_barrier()

    def wout(k, carry):
        j = lax.rem(k, 4)

        @pl.when(k >= 4)
        def _drain():
            pltpu.make_async_copy(rows_b.at[j],
                                  acc_out.at[c, pl.ds(0, CHUNK)],
                                  sem.at[j]).wait()

        rk = pl.multiple_of(r0 + k * CHUNK, CHUNK)
        pltpu.sync_copy(acc_sh.at[pl.ds(rk, CHUNK)], rows_b.at[j])
        pltpu.async_copy(rows_b.at[j], acc_out.at[c, pl.ds(rk, CHUNK)],
                         sem.at[j])
        return carry

    lax.fori_loop(0, N_STAGE, wout, 0)

    def wdrain(t, carry):
        jt = lax.rem(N_STAGE - 1 - t, 4)
        pltpu.make_async_copy(rows_b.at[jt], acc_out.at[c, pl.ds(0, CHUNK)],
                              sem.at[jt]).wait()
        return carry

    lax.fori_loop(0, 4, wdrain, 0)


def _deg_body(dst_hbm, zfeat, ones_h, deg_out, deg_sh, dst_b, ones_v, sem):
    c = lax.axis_index("c")
    s = lax.axis_index("s")
    g = c * NS + s
    r0 = s * ROWS_PER_TILE

    def zinit(k, carry):
        rk = pl.multiple_of(r0 + k * CHUNK, CHUNK)
        pltpu.sync_copy(zfeat.at[pl.ds(rk, CHUNK)], ones_v)
        pltpu.sync_copy(ones_v, deg_sh.at[pl.ds(rk, CHUNK)])
        return carry

    lax.fori_loop(0, N_STAGE, zinit, 0)
    pltpu.sync_copy(ones_h, ones_v)
    plsc.subcore_barrier()
    base = g * EPT_PAD

    def body(i, carry):
        j = lax.rem(i, 3)

        @pl.when(i >= 3)
        def _drain():
            pltpu.make_async_copy(ones_v, deg_sh.at[dst_b.at[j]],
                                  sem.at[j]).wait()

        off = pl.multiple_of(base + i * CHUNK, CHUNK)
        pltpu.sync_copy(dst_hbm.at[pl.ds(off, CHUNK)], dst_b.at[j])
        pltpu.async_copy(ones_v, deg_sh.at[dst_b.at[j]], sem.at[j], add=True)
        return carry

    lax.fori_loop(0, N_CHUNKS, body, 0)

    def drain(j, carry):
        pltpu.make_async_copy(ones_v, deg_sh.at[dst_b.at[j]],
                              sem.at[j]).wait()
        return carry

    lax.fori_loop(0, 3, drain, 0)
    plsc.subcore_barrier()

    def wout(k, carry):
        rk = pl.multiple_of(r0 + k * CHUNK, CHUNK)
        pltpu.sync_copy(deg_sh.at[pl.ds(rk, CHUNK)], ones_v)
        pltpu.sync_copy(ones_v, deg_out.at[c, pl.ds(rk, CHUNK)])
        return carry

    lax.fori_loop(0, N_STAGE, wout, 0)


_MESH = plsc.VectorSubcoreMesh(core_axis_name="c", subcore_axis_name="s")

_agg = pl.kernel(
    _agg_body,
    out_type=[jax.ShapeDtypeStruct((NC, N_PAD, D), jnp.float32)],
    mesh=_MESH,
    scratch_types=[
        pltpu.VMEM_SHARED((N_PAD, D), jnp.float32),
        pltpu.VMEM((4, CHUNK), jnp.int32),
        pltpu.VMEM((4, CHUNK), jnp.int32),
        pltpu.VMEM((4, CHUNK, D), jnp.float32),
        pltpu.SemaphoreType.DMA((4,)),
        pltpu.SemaphoreType.DMA((4,)),
    ],
)

_deg_count = pl.kernel(
    _deg_body,
    out_type=[jax.ShapeDtypeStruct((NC, N_PAD, D), jnp.float32)],
    mesh=_MESH,
    scratch_types=[
        pltpu.VMEM_SHARED((N_PAD, D), jnp.float32),
        pltpu.VMEM((3, CHUNK), jnp.int32),
        pltpu.VMEM((CHUNK, D), jnp.float32),
        pltpu.SemaphoreType.DMA((3,)),
    ],
)


BLK = 1000


def _lin2_body(x_ref, wn_ref, ws_ref, b_ref, p_ref, s_ref):
    xv = x_ref[...]
    p_ref[...] = jnp.dot(xv, wn_ref[...], preferred_element_type=jnp.float32)
    s_ref[...] = (jnp.dot(xv, ws_ref[...], preferred_element_type=jnp.float32)
                  + b_ref[...])


def _combine_lin2_body(a0, a1, d0, d1, s_in, wn, ws, b, p_ref, s_ref):
    deg = d0[:, 0:1] + d1[:, 0:1]
    inv = 1.0 / jnp.maximum(deg, 1.0)
    h = jnp.maximum((a0[...] + a1[...]) * inv + s_in[...], 0.0)
    p_ref[...] = jnp.dot(h, wn[...], preferred_element_type=jnp.float32)
    s_ref[...] = (jnp.dot(h, ws[...], preferred_element_type=jnp.float32)
                  + b[...])


def _combine_fc_body(a0, a1, d0, d1, s_in, wfc, bfc, o_ref):
    deg = d0[:, 0:1] + d1[:, 0:1]
    inv = 1.0 / jnp.maximum(deg, 1.0)
    h = jnp.maximum((a0[...] + a1[...]) * inv + s_in[...], 0.0)
    o_ref[...] = (jnp.dot(h, wfc[...], preferred_element_type=jnp.float32)
                  + bfc[...])


def _row_spec(w=D):
    return pl.BlockSpec((BLK, w), lambda i: (i, 0))


def _w_spec():
    return pl.BlockSpec((D, D), lambda i: (0, 0))


def _b_spec():
    return pl.BlockSpec((1, D), lambda i: (0, 0))


_GRID = N_NODES // BLK

_lin2 = pl.pallas_call(
    _lin2_body,
    grid=(_GRID,),
    in_specs=[_row_spec(), _w_spec(), _w_spec(), _b_spec()],
    out_specs=[_row_spec(), _row_spec()],
    out_shape=[jax.ShapeDtypeStruct((N_NODES, D), jnp.float32)] * 2,
)

_combine_lin2 = pl.pallas_call(
    _combine_lin2_body,
    grid=(_GRID,),
    in_specs=[_row_spec(), _row_spec(), _row_spec(), _row_spec(),
              _row_spec(), _w_spec(), _w_spec(), _b_spec()],
    out_specs=[_row_spec(), _row_spec()],
    out_shape=[jax.ShapeDtypeStruct((N_NODES, D), jnp.float32)] * 2,
)

_combine_fc = pl.pallas_call(
    _combine_fc_body,
    grid=(_GRID,),
    in_specs=[_row_spec(), _row_spec(), _row_spec(), _row_spec(),
              _row_spec(), _w_spec(), _b_spec()],
    out_specs=_row_spec(),
    out_shape=jax.ShapeDtypeStruct((N_NODES, D), jnp.float32),
)


@jax.jit
def kernel(x, edge_index, W_n1, W_s1, b1, W_n2, W_s2, b2, W_fc, b_fc):
    pad_src = jnp.zeros((E_PAD - N_EDGES,), jnp.int32)
    pad_dst = jnp.full((E_PAD - N_EDGES,), TRASH_ROW, jnp.int32)
    src = jnp.concatenate([edge_index[0], pad_src])
    dst = jnp.concatenate([edge_index[1], pad_dst])
    zfeat = jnp.zeros((N_PAD, D), jnp.float32)
    ones_h = jnp.ones((CHUNK, D), jnp.float32)
    b1r = b1.reshape(1, D)
    b2r = b2.reshape(1, D)
    bfr = b_fc.reshape(1, D)

    p1, s1 = _lin2(x, W_n1, W_s1, b1r)
    degp, = _deg_count(dst, zfeat, ones_h)
    p1 = p1 + jnp.minimum(degp[0, :N_NODES], 0.0)
    acc1, = _agg(p1, src, dst, zfeat)
    p2, s2 = _combine_lin2(acc1[0], acc1[1], degp[0], degp[1], s1,
                           W_n2, W_s2, b2r)
    acc2, = _agg(p2, src, dst, zfeat)
    out = _combine_fc(acc2[0], acc2[1], degp[0], degp[1], s2, W_fc, bfr)
    return out


# speedup vs baseline: 1.0669x; 1.0669x over previous
import functools

import jax
import jax.numpy as jnp
from jax import lax
from jax.experimental import pallas as pl
from jax.experimental.pallas import tpu as pltpu
from jax.experimental.pallas import tpu_sc as plsc

N_NODES = 10000
D = 128
N_EDGES = 320000

NC = 2
NS = 16
N_TILES = NC * NS
CHUNK = 64
N_CHUNKS = 160
EPT_PAD = N_CHUNKS * CHUNK
E_PAD = EPT_PAD * N_TILES
ROWS_PER_TILE = 640
N_STAGE = 10
N_PAD = ROWS_PER_TILE * NS
TRASH_ROW = N_NODES
DEG_W = 16


def _agg_body(p_hbm, src_hbm, dst_hbm, zfeat,
              acc_out, acc_sh, src_b, dst_b, rows_b, sem, sem2):
    c = lax.axis_index("c")
    s = lax.axis_index("s")
    g = c * NS + s
    r0 = s * ROWS_PER_TILE

    pltpu.sync_copy(zfeat.at[pl.ds(pl.multiple_of(r0, CHUNK), CHUNK)],
                    rows_b.at[0])

    def zinit(k, carry):
        rk = pl.multiple_of(r0 + k * CHUNK, CHUNK)
        pltpu.async_copy(rows_b.at[0], acc_sh.at[pl.ds(rk, CHUNK)],
                         sem2.at[0])
        return carry

    lax.fori_loop(0, N_STAGE, zinit, 0)

    def zdrain(k, carry):
        pltpu.make_async_copy(rows_b.at[0],
                              acc_sh.at[pl.ds(pl.multiple_of(r0, CHUNK),
                                              CHUNK)], sem2.at[0]).wait()
        return carry

    lax.fori_loop(0, N_STAGE, zdrain, 0)
    plsc.subcore_barrier()
    base = g * EPT_PAD

    pltpu.sync_copy(src_hbm.at[pl.ds(pl.multiple_of(base, CHUNK), CHUNK)],
                    src_b.at[0])
    pltpu.async_copy(p_hbm.at[src_b.at[0]], rows_b.at[0], sem.at[0])

    def body(i, carry):
        j = lax.rem(i, 3)
        jn = lax.rem(i + 1, 3)

        @pl.when(i >= 2)
        def _drain_scatter():
            pltpu.make_async_copy(rows_b.at[jn], acc_sh.at[dst_b.at[jn]],
                                  sem2.at[jn]).wait()

        @pl.when(i < N_CHUNKS - 1)
        def _prefetch():
            off = pl.multiple_of(base + (i + 1) * CHUNK, CHUNK)
            pltpu.sync_copy(src_hbm.at[pl.ds(off, CHUNK)], src_b.at[jn])
            pltpu.async_copy(p_hbm.at[src_b.at[jn]], rows_b.at[jn], sem.at[jn])

        off_d = pl.multiple_of(base + i * CHUNK, CHUNK)
        pltpu.sync_copy(dst_hbm.at[pl.ds(off_d, CHUNK)], dst_b.at[j])
        pltpu.make_async_copy(p_hbm.at[src_b.at[j]], rows_b.at[j], sem.at[j]).wait()
        pltpu.async_copy(rows_b.at[j], acc_sh.at[dst_b.at[j]], sem2.at[j], add=True)
        return carry

    lax.fori_loop(0, N_CHUNKS, body, 0)
    last = lax.rem(N_CHUNKS - 1, 3)
    prev = lax.rem(N_CHUNKS - 2, 3)
    pltpu.make_async_copy(rows_b.at[last], acc_sh.at[dst_b.at[last]],
                          sem2.at[last]).wait()
    pltpu.make_async_copy(rows_b.at[prev], acc_sh.at[dst_b.at[prev]],
                          sem2.at[prev]).wait()
    plsc.subcore_barrier()

    def wout(k, carry):
        j = lax.rem(k, 3)

        @pl.when(k >= 3)
        def _drain():
            pltpu.make_async_copy(rows_b.at[j],
                                  acc_out.at[c, pl.ds(0, CHUNK)],
                                  sem.at[j]).wait()

        rk = pl.multiple_of(r0 + k * CHUNK, CHUNK)
        pltpu.sync_copy(acc_sh.at[pl.ds(rk, CHUNK)], rows_b.at[j])
        pltpu.async_copy(rows_b.at[j], acc_out.at[c, pl.ds(rk, CHUNK)],
                         sem.at[j])
        return carry

    lax.fori_loop(0, N_STAGE, wout, 0)

    def wdrain(j, carry):
        pltpu.make_async_copy(rows_b.at[j], acc_out.at[c, pl.ds(0, CHUNK)],
                              sem.at[j]).wait()
        return carry

    lax.fori_loop(0, 3, wdrain, 0)


DCH = 128
DN_CHUNKS = EPT_PAD // DCH
DN_STAGE = ROWS_PER_TILE // DCH


def _deg_body(dst_hbm, zfeat, ones_h, deg_out, deg_sh, dst_b, ones_v, sem):
    c = lax.axis_index("c")
    s = lax.axis_index("s")
    g = c * NS + s
    r0 = s * ROWS_PER_TILE

    def zinit(k, carry):
        rk = pl.multiple_of(r0 + k * DCH, DCH)
        pltpu.sync_copy(zfeat.at[pl.ds(rk, DCH)], ones_v)
        pltpu.sync_copy(ones_v, deg_sh.at[pl.ds(rk, DCH)])
        return carry

    lax.fori_loop(0, DN_STAGE, zinit, 0)
    pltpu.sync_copy(ones_h, ones_v)
    plsc.subcore_barrier()
    base = g * EPT_PAD

    def body(i, carry):
        j = lax.rem(i, 3)

        @pl.when(i >= 3)
        def _drain():
            pltpu.make_async_copy(ones_v, deg_sh.at[dst_b.at[j]],
                                  sem.at[j]).wait()

        off = pl.multiple_of(base + i * DCH, DCH)
        pltpu.sync_copy(dst_hbm.at[pl.ds(off, DCH)], dst_b.at[j])
        pltpu.async_copy(ones_v, deg_sh.at[dst_b.at[j]], sem.at[j], add=True)
        return carry

    lax.fori_loop(0, DN_CHUNKS, body, 0)

    def drain(j, carry):
        pltpu.make_async_copy(ones_v, deg_sh.at[dst_b.at[j]],
                              sem.at[j]).wait()
        return carry

    lax.fori_loop(0, 3, drain, 0)
    plsc.subcore_barrier()

    def wout(k, carry):
        rk = pl.multiple_of(r0 + k * DCH, DCH)
        pltpu.sync_copy(deg_sh.at[pl.ds(rk, DCH)], ones_v)
        pltpu.sync_copy(ones_v, deg_out.at[c, pl.ds(rk, DCH)])
        return carry

    lax.fori_loop(0, DN_STAGE, wout, 0)


_MESH = plsc.VectorSubcoreMesh(core_axis_name="c", subcore_axis_name="s")

_agg = pl.kernel(
    _agg_body,
    out_type=[jax.ShapeDtypeStruct((NC, N_PAD, D), jnp.float32)],
    mesh=_MESH,
    scratch_types=[
        pltpu.VMEM_SHARED((N_PAD, D), jnp.float32),
        pltpu.VMEM((3, CHUNK), jnp.int32),
        pltpu.VMEM((3, CHUNK), jnp.int32),
        pltpu.VMEM((3, CHUNK, D), jnp.float32),
        pltpu.SemaphoreType.DMA((3,)),
        pltpu.SemaphoreType.DMA((3,)),
    ],
)

_deg_count = pl.kernel(
    _deg_body,
    out_type=[jax.ShapeDtypeStruct((NC, N_PAD, D), jnp.float32)],
    mesh=_MESH,
    scratch_types=[
        pltpu.VMEM_SHARED((N_PAD, D), jnp.float32),
        pltpu.VMEM((3, DCH), jnp.int32),
        pltpu.VMEM((DCH, D), jnp.float32),
        pltpu.SemaphoreType.DMA((3,)),
    ],
)


BLK = 1000


def _lin2_body(x_ref, wn_ref, ws_ref, b_ref, p_ref, s_ref):
    xv = x_ref[...]
    p_ref[...] = jnp.dot(xv, wn_ref[...], preferred_element_type=jnp.float32)
    s_ref[...] = (jnp.dot(xv, ws_ref[...], preferred_element_type=jnp.float32)
                  + b_ref[...])


def _combine_lin2_body(a0, a1, d0, d1, s_in, wn, ws, b, p_ref, s_ref):
    deg = d0[:, 0:1] + d1[:, 0:1]
    inv = 1.0 / jnp.maximum(deg, 1.0)
    h = jnp.maximum((a0[...] + a1[...]) * inv + s_in[...], 0.0)
    p_ref[...] = jnp.dot(h, wn[...], preferred_element_type=jnp.float32)
    s_ref[...] = (jnp.dot(h, ws[...], preferred_element_type=jnp.float32)
                  + b[...])


def _combine_fc_body(a0, a1, d0, d1, s_in, wfc, bfc, o_ref):
    deg = d0[:, 0:1] + d1[:, 0:1]
    inv = 1.0 / jnp.maximum(deg, 1.0)
    h = jnp.maximum((a0[...] + a1[...]) * inv + s_in[...], 0.0)
    o_ref[...] = (jnp.dot(h, wfc[...], preferred_element_type=jnp.float32)
                  + bfc[...])


def _row_spec(w=D):
    return pl.BlockSpec((BLK, w), lambda i: (i, 0))


def _w_spec():
    return pl.BlockSpec((D, D), lambda i: (0, 0))


def _b_spec():
    return pl.BlockSpec((1, D), lambda i: (0, 0))


_GRID = N_NODES // BLK

_lin2 = pl.pallas_call(
    _lin2_body,
    grid=(_GRID,),
    in_specs=[_row_spec(), _w_spec(), _w_spec(), _b_spec()],
    out_specs=[_row_spec(), _row_spec()],
    out_shape=[jax.ShapeDtypeStruct((N_NODES, D), jnp.float32)] * 2,
)

_combine_lin2 = pl.pallas_call(
    _combine_lin2_body,
    grid=(_GRID,),
    in_specs=[_row_spec(), _row_spec(), _row_spec(), _row_spec(),
              _row_spec(), _w_spec(), _w_spec(), _b_spec()],
    out_specs=[_row_spec(), _row_spec()],
    out_shape=[jax.ShapeDtypeStruct((N_NODES, D), jnp.float32)] * 2,
)

_combine_fc = pl.pallas_call(
    _combine_fc_body,
    grid=(_GRID,),
    in_specs=[_row_spec(), _row_spec(), _row_spec(), _row_spec(),
              _row_spec(), _w_spec(), _b_spec()],
    out_specs=_row_spec(),
    out_shape=jax.ShapeDtypeStruct((N_NODES, D), jnp.float32),
)


@jax.jit
def kernel(x, edge_index, W_n1, W_s1, b1, W_n2, W_s2, b2, W_fc, b_fc):
    pad_src = jnp.zeros((E_PAD - N_EDGES,), jnp.int32)
    pad_dst = jnp.full((E_PAD - N_EDGES,), TRASH_ROW, jnp.int32)
    src = jnp.concatenate([edge_index[0], pad_src])
    dst = jnp.concatenate([edge_index[1], pad_dst])
    zfeat = jnp.zeros((N_PAD, D), jnp.float32)
    ones_h = jnp.ones((DCH, D), jnp.float32)
    b1r = b1.reshape(1, D)
    b2r = b2.reshape(1, D)
    bfr = b_fc.reshape(1, D)

    p1, s1 = _lin2(x, W_n1, W_s1, b1r)
    degp, = _deg_count(dst, zfeat, ones_h)
    p1 = p1 + jnp.minimum(degp[0, :N_NODES], 0.0)
    acc1, = _agg(p1, src, dst, zfeat)
    p2, s2 = _combine_lin2(acc1[0], acc1[1], degp[0], degp[1], s1,
                           W_n2, W_s2, b2r)
    acc2, = _agg(p2, src, dst, zfeat)
    out = _combine_fc(acc2[0], acc2[1], degp[0], degp[1], s2, W_fc, bfr)
    return out
